# baseline (speedup 1.0000x reference)
import functools

import jax
import jax.numpy as jnp
from jax import lax
from jax.experimental import pallas as pl
from jax.experimental.pallas import tpu as pltpu
from jax.experimental.pallas import tpu_sc as plsc

N = 10000
D = 128
E = 320000
NP = 10240
EP = 327680
CHUNK = 128
NCHUNK = EP // CHUNK
NW = 32
CPW = NCHUNK // NW
EPW = EP // NW
ROWS_PT = NP // 16
RB = NP // 1024

_mesh = plsc.VectorSubcoreMesh(core_axis_name="c", subcore_axis_name="s")


@functools.partial(
    pl.kernel,
    out_type=jax.ShapeDtypeStruct((NP // 16, 16), jnp.float32),
    mesh=_mesh,
    compiler_params=pltpu.CompilerParams(needs_layout_passes=False, use_tc_tiling_on_sc=False),
    scratch_types=[
        pltpu.VMEM((EP // 16,), jnp.int32),
        pltpu.VMEM((NP // 32, 16), jnp.float32),
        pltpu.VMEM((NP // 512, 16), jnp.float32),
        pltpu.VMEM((NP // 512, 16), jnp.float32),
        pltpu.VMEM_SHARED((16, NP // 32, 16), jnp.float32),
    ],
)
def _sc_degree(col_hbm, out_hbm, colbuf, acc2d, fbuf, tbuf, sh16):
    c = lax.axis_index("c")
    s = lax.axis_index("s")
    ept = EP // 16
    rph = NP // 32
    rpt = NP // 512
    pltpu.sync_copy(col_hbm.at[pl.ds(s * ept, ept)], colbuf)
    zeros = jnp.zeros((16,), jnp.float32)

    def zbody(i, _):
        acc2d[i, pl.ds(0, 16)] = zeros
        return 0

    lax.fori_loop(0, rph, zbody, 0)
    ones = jnp.ones((16,), jnp.float32)
    half_base = c * (NP // 2)

    def ebody(i, _):
        cc = colbuf[pl.ds(i * 16, 16)]
        lc = cc - half_base
        valid = (lc >= 0) & (lc < NP // 2)
        slc = jnp.where(valid, lc, 0)
        plsc.addupdate_scatter(
            acc2d,
            [lax.shift_right_logical(slc, 4), lax.bitwise_and(slc, 15)],
            ones, mask=valid)
        return 0

    lax.fori_loop(0, ept // 16, ebody, 0)
    pltpu.sync_copy(acc2d, sh16.at[s])
    plsc.subcore_barrier()
    rbase = s * rpt
    pltpu.sync_copy(sh16.at[0, pl.ds(rbase, rpt)], fbuf)
    for t in range(1, 16):
        pltpu.sync_copy(sh16.at[t, pl.ds(rbase, rpt)], tbuf)
        for g in range(rpt):
            fbuf[g, pl.ds(0, 16)] = (fbuf[g, pl.ds(0, 16)]
                                     + tbuf[g, pl.ds(0, 16)])
    pltpu.sync_copy(fbuf, out_hbm.at[pl.ds(c * rph + rbase, rpt)])


DH = D // 2
CPT = NCHUNK // 16
CS = 1


NB = 5
QC = 20


@functools.partial(
    pl.kernel,
    out_type=jax.ShapeDtypeStruct((2, NP, DH), jnp.float32),
    mesh=_mesh,
    compiler_params=pltpu.CompilerParams(needs_layout_passes=False, use_tc_tiling_on_sc=False),
    scratch_types=[
        pltpu.VMEM((QC, CHUNK), jnp.int32),
        pltpu.VMEM((QC, CHUNK), jnp.int32),
        [pltpu.VMEM((CHUNK, DH), jnp.float32)] * NB,
        pltpu.VMEM_SHARED((NP, DH), jnp.float32),
        pltpu.VMEM_SHARED((NP, DH), jnp.float32),
        [pltpu.SemaphoreType.DMA] * NB,
        [pltpu.SemaphoreType.DMA] * NB,
    ],
)
def _sc_agg(hs_hbm, row_hbm, col_hbm, out_hbm, rowbuf, colbuf, gbufs,
            hssh, shacc, semg, sems):
    gb0 = gbufs[0]
    gb1 = gbufs[1]
    c = lax.axis_index("c")
    s = lax.axis_index("s")

    zeros = jnp.zeros((16,), jnp.float32)

    def zb(i, _):
        r = i // 4
        k = i - r * 4
        gb0[r, pl.ds(k * 16, 16)] = zeros
        return 0

    lax.fori_loop(0, CHUNK * (DH // 16), zb, 0)
    pltpu.sync_copy(hs_hbm.at[c].at[pl.ds(s * ROWS_PT, ROWS_PT)],
                    hssh.at[pl.ds(s * ROWS_PT, ROWS_PT)])
    for t in range(ROWS_PT // CHUNK):
        base = s * ROWS_PT + t * CHUNK
        pltpu.sync_copy(gb0, shacc.at[pl.ds(base, CHUNK)])
    plsc.subcore_barrier()

    for q in range(CPT // QC):
        qbase = s * CPT + q * QC
        pltpu.sync_copy(row_hbm.at[pl.ds(qbase, QC)], rowbuf)
        pltpu.sync_copy(col_hbm.at[pl.ds(qbase, QC)], colbuf)
        for b in range(NB):
            pltpu.async_copy(hssh.at[rowbuf.at[b]], gbufs[b], semg[b])

        @pl.loop(0, QC, step=NB)
        def ring(j):
            descs = []
            for b in range(NB):
                pltpu.make_async_copy(
                    hssh.at[rowbuf.at[j + b]], gbufs[b], semg[b]).wait()
                descs.append(pltpu.async_copy(
                    gbufs[b], shacc.at[colbuf.at[j + b]], sems[b], add=True))
            for b in range(NB):
                descs[b].wait()

                @pl.when(j + b + NB < QC)
                def _():
                    pltpu.async_copy(
                        hssh.at[rowbuf.at[j + b + NB]], gbufs[b], semg[b])

    plsc.subcore_barrier()
    pltpu.sync_copy(shacc.at[pl.ds(s * ROWS_PT, ROWS_PT)],
                    out_hbm.at[c, pl.ds(s * ROWS_PT, ROWS_PT)])


NH = NP // 2
RPH = NH // 16
EPT = EP // 16
RPT = RPH // 16


@functools.partial(
    pl.kernel,
    out_type=jax.ShapeDtypeStruct((NP // 16, 16), jnp.float32),
    mesh=_mesh,
    compiler_params=pltpu.CompilerParams(needs_layout_passes=False, use_tc_tiling_on_sc=False),
    scratch_types=[
        pltpu.VMEM((NP,), jnp.float32),
        pltpu.VMEM((EPT,), jnp.int32),
        pltpu.VMEM((EPT,), jnp.int32),
        pltpu.VMEM((RPH, 16), jnp.float32),
        pltpu.VMEM((RPT, 16), jnp.float32),
        pltpu.VMEM((RPT, 16), jnp.float32),
        pltpu.VMEM_SHARED((16, RPH, 16), jnp.float32),
    ],
)
def _sc_seg2(ts_hbm, row_hbm, col_hbm, dinv_hbm, x0bc_hbm, out_hbm,
             tsbuf, rowbuf, colbuf, acc2d, fbuf, tbuf, sh16):
    c = lax.axis_index("c")
    s = lax.axis_index("s")
    pltpu.sync_copy(ts_hbm, tsbuf)
    pltpu.sync_copy(row_hbm.at[pl.ds(s * EPT, EPT)], rowbuf)
    pltpu.sync_copy(col_hbm.at[pl.ds(s * EPT, EPT)], colbuf)
    zeros = jnp.zeros((16,), jnp.float32)

    def zbody(i, _):
        acc2d[i, pl.ds(0, 16)] = zeros
        return 0

    lax.fori_loop(0, RPH, zbody, 0)

    half_base = c * NH

    def ebody(i, _):
        sl = pl.ds(i * 16, 16)
        r = rowbuf[sl]
        cc = colbuf[sl]
        v = plsc.load_gather(tsbuf, [r])
        lc = cc - half_base
        valid = (lc >= 0) & (lc < NH)
        slc = jnp.where(valid, lc, 0)
        plsc.addupdate_scatter(
            acc2d,
            [lax.shift_right_logical(slc, 4), lax.bitwise_and(slc, 15)],
            v, mask=valid)
        return 0

    lax.fori_loop(0, EPT // 16, ebody, 0)

    pltpu.sync_copy(acc2d, sh16.at[s])
    plsc.subcore_barrier()
    rbase = s * RPT
    pltpu.sync_copy(sh16.at[0, pl.ds(rbase, RPT)], fbuf)
    for t in range(1, 16):
        pltpu.sync_copy(sh16.at[t, pl.ds(rbase, RPT)], tbuf)
        for g in range(RPT):
            fbuf[g, pl.ds(0, 16)] = (fbuf[g, pl.ds(0, 16)]
                                     + tbuf[g, pl.ds(0, 16)])
    gbase = c * RPH + rbase
    pltpu.sync_copy(dinv_hbm.at[pl.ds(gbase, RPT)], tbuf)
    for g in range(RPT):
        fbuf[g, pl.ds(0, 16)] = fbuf[g, pl.ds(0, 16)] * tbuf[g, pl.ds(0, 16)]
    pltpu.sync_copy(x0bc_hbm.at[pl.ds(gbase, RPT)], tbuf)
    for g in range(RPT):
        fbuf[g, pl.ds(0, 16)] = fbuf[g, pl.ds(0, 16)] + tbuf[g, pl.ds(0, 16)]
    pltpu.sync_copy(fbuf, out_hbm.at[pl.ds(gbase, RPT)])


def _tc_dense1(x_ref, c1w_ref, f1w_ref, f1b_ref, degp_ref,
               hs_ref, x0_ref, dinv_ref):
    deg = degp_ref[...]
    dinv = jnp.where(deg > 0, lax.rsqrt(deg), 0.0)
    x = x_ref[...]
    h = jnp.dot(x, c1w_ref[...], preferred_element_type=jnp.float32)
    hsc = h * dinv
    hs_ref[0] = hsc[:, :DH]
    hs_ref[1] = hsc[:, DH:]
    x0_ref[...] = jnp.maximum(
        jnp.dot(x, f1w_ref[...], preferred_element_type=jnp.float32)
        + f1b_ref[...], 0.0)
    dinv_ref[...] = dinv


def _tc_mid(acc_ref, x0_ref, dinv_ref, c1b_ref, c2w_ref, f2w_ref, f2b_ref,
            c2b_ref, ts_ref, x0b_ref):
    agg = jnp.concatenate([acc_ref[0], acc_ref[1]], axis=-1)
    dinv = dinv_ref[...]
    conv1 = jnp.maximum(agg * dinv + c1b_ref[...], 0.0)
    xsum = x0_ref[...] + conv1
    ts_ref[...] = jnp.dot(xsum, c2w_ref[...],
                          preferred_element_type=jnp.float32) * dinv
    x0b_ref[...] = (jnp.dot(xsum, f2w_ref[...],
                            preferred_element_type=jnp.float32)
                    + f2b_ref[...] + c2b_ref[...])


def kernel(x, edge_index, conv1_W, conv1_b, conv2_W, conv2_b,
           fc1_W, fc1_b, fc2_W, fc2_b):
    f32 = jnp.float32
    xp = jnp.pad(x, ((0, NP - N), (0, 0)))
    fill = jnp.full((EP - E,), NP - 1, dtype=jnp.int32)
    rowp = jnp.concatenate([edge_index[0], fill])
    colp = jnp.concatenate([edge_index[1], fill])
    row2d = rowp.reshape(NCHUNK, CHUNK)
    col2d = colp.reshape(NCHUNK, CHUNK)

    degp = _sc_degree(colp).reshape(NP, 1)

    hs, x0, dinv = pl.pallas_call(
        _tc_dense1,
        grid=(RB,),
        in_specs=[
            pl.BlockSpec((1024, D), lambda i: (i, 0)),
            pl.BlockSpec((D, D), lambda i: (0, 0)),
            pl.BlockSpec((D, D), lambda i: (0, 0)),
            pl.BlockSpec((1, D), lambda i: (0, 0)),
            pl.BlockSpec((1024, 1), lambda i: (i, 0)),
        ],
        out_specs=[
            pl.BlockSpec((2, 1024, DH), lambda i: (0, i, 0)),
            pl.BlockSpec((1024, D), lambda i: (i, 0)),
            pl.BlockSpec((1024, 1), lambda i: (i, 0)),
        ],
        out_shape=[
            jax.ShapeDtypeStruct((2, NP, DH), f32),
            jax.ShapeDtypeStruct((NP, D), f32),
            jax.ShapeDtypeStruct((NP, 1), f32),
        ],
    )(xp, conv1_W, fc1_W, fc1_b.reshape(1, D), degp)

    acc2 = _sc_agg(hs, row2d, col2d)

    ts, x0b = pl.pallas_call(
        _tc_mid,
        grid=(RB,),
        in_specs=[
            pl.BlockSpec((2, 1024, DH), lambda i: (0, i, 0)),
            pl.BlockSpec((1024, D), lambda i: (i, 0)),
            pl.BlockSpec((1024, 1), lambda i: (i, 0)),
            pl.BlockSpec((1, D), lambda i: (0, 0)),
            pl.BlockSpec((D, 1), lambda i: (0, 0)),
            pl.BlockSpec((D, 1), lambda i: (0, 0)),
            pl.BlockSpec((1, 1), lambda i: (0, 0)),
            pl.BlockSpec((1, 1), lambda i: (0, 0)),
        ],
        out_specs=[
            pl.BlockSpec((1024, 1), lambda i: (i, 0)),
            pl.BlockSpec((1024, 1), lambda i: (i, 0)),
        ],
        out_shape=[
            jax.ShapeDtypeStruct((NP, 1), f32),
            jax.ShapeDtypeStruct((NP, 1), f32),
        ],
    )(acc2, x0, dinv, conv1_b.reshape(1, D), conv2_W, fc2_W,
      fc2_b.reshape(1, 1), conv2_b.reshape(1, 1))

    out2d = _sc_seg2(ts.reshape(NP), rowp, colp,
                     dinv.reshape(NP // 16, 16), x0b.reshape(NP // 16, 16))
    return out2d.reshape(NP, 1)[:N]


# speedup vs baseline: 1.0090x; 1.0090x over previous
import functools

import jax
import jax.numpy as jnp
from jax import lax
from jax.experimental import pallas as pl
from jax.experimental.pallas import tpu as pltpu
from jax.experimental.pallas import tpu_sc as plsc

N = 10000
D = 128
E = 320000
NP = 10240
EP = 327680
CHUNK = 128
NCHUNK = EP // CHUNK
NW = 32
CPW = NCHUNK // NW
EPW = EP // NW
ROWS_PT = NP // 16
RB = NP // 1024

_mesh = plsc.VectorSubcoreMesh(core_axis_name="c", subcore_axis_name="s")


@functools.partial(
    pl.kernel,
    out_type=jax.ShapeDtypeStruct((NP // 16, 16), jnp.float32),
    mesh=_mesh,
    compiler_params=pltpu.CompilerParams(needs_layout_passes=False, use_tc_tiling_on_sc=False),
    scratch_types=[
        pltpu.VMEM((EP // 16,), jnp.int32),
        pltpu.VMEM((NP // 32, 16), jnp.float32),
        pltpu.VMEM((NP // 512, 16), jnp.float32),
        pltpu.VMEM((NP // 512, 16), jnp.float32),
        pltpu.VMEM_SHARED((16, NP // 32, 16), jnp.float32),
    ],
)
def _sc_degree(col_hbm, out_hbm, colbuf, acc2d, fbuf, tbuf, sh16):
    c = lax.axis_index("c")
    s = lax.axis_index("s")
    ept = EP // 16
    rph = NP // 32
    rpt = NP // 512
    pltpu.sync_copy(col_hbm.at[pl.ds(s * ept, ept)], colbuf)
    zeros = jnp.zeros((16,), jnp.float32)

    def zbody(i, _):
        acc2d[i, pl.ds(0, 16)] = zeros
        return 0

    lax.fori_loop(0, rph, zbody, 0)
    ones = jnp.ones((16,), jnp.float32)
    half_base = c * (NP // 2)

    def ebody(i, _):
        cc = colbuf[pl.ds(i * 16, 16)]
        lc = cc - half_base
        valid = (lc >= 0) & (lc < NP // 2)
        slc = jnp.where(valid, lc, 0)
        plsc.addupdate_scatter(
            acc2d,
            [lax.shift_right_logical(slc, 4), lax.bitwise_and(slc, 15)],
            ones, mask=valid)
        return 0

    lax.fori_loop(0, ept // 16, ebody, 0)
    pltpu.sync_copy(acc2d, sh16.at[s])
    plsc.subcore_barrier()
    rbase = s * rpt
    pltpu.sync_copy(sh16.at[0, pl.ds(rbase, rpt)], fbuf)
    for t in range(1, 16):
        pltpu.sync_copy(sh16.at[t, pl.ds(rbase, rpt)], tbuf)
        for g in range(rpt):
            fbuf[g, pl.ds(0, 16)] = (fbuf[g, pl.ds(0, 16)]
                                     + tbuf[g, pl.ds(0, 16)])
    pltpu.sync_copy(fbuf, out_hbm.at[pl.ds(c * rph + rbase, rpt)])


DH = D // 2
CPT = NCHUNK // 16
CS = 1


NB = 4
QC = 40


@functools.partial(
    pl.kernel,
    out_type=jax.ShapeDtypeStruct((2, NP, DH), jnp.float32),
    mesh=_mesh,
    compiler_params=pltpu.CompilerParams(needs_layout_passes=False, use_tc_tiling_on_sc=False),
    scratch_types=[
        pltpu.VMEM((QC, CHUNK), jnp.int32),
        pltpu.VMEM((QC, CHUNK), jnp.int32),
        [pltpu.VMEM((CHUNK, DH), jnp.float32)] * NB,
        pltpu.VMEM_SHARED((NP, DH), jnp.float32),
        pltpu.VMEM_SHARED((NP, DH), jnp.float32),
        [pltpu.SemaphoreType.DMA] * NB,
        [pltpu.SemaphoreType.DMA] * NB,
    ],
)
def _sc_agg(hs_hbm, row_hbm, col_hbm, out_hbm, rowbuf, colbuf, gbufs,
            hssh, shacc, semg, sems):
    gb0 = gbufs[0]
    gb1 = gbufs[1]
    c = lax.axis_index("c")
    s = lax.axis_index("s")

    zeros = jnp.zeros((16,), jnp.float32)

    def zb(i, _):
        r = i // 4
        k = i - r * 4
        gb0[r, pl.ds(k * 16, 16)] = zeros
        return 0

    lax.fori_loop(0, CHUNK * (DH // 16), zb, 0)
    pltpu.sync_copy(hs_hbm.at[c].at[pl.ds(s * ROWS_PT, ROWS_PT)],
                    hssh.at[pl.ds(s * ROWS_PT, ROWS_PT)])
    for t in range(ROWS_PT // CHUNK):
        base = s * ROWS_PT + t * CHUNK
        pltpu.sync_copy(gb0, shacc.at[pl.ds(base, CHUNK)])
    plsc.subcore_barrier()

    for q in range(CPT // QC):
        qbase = s * CPT + q * QC
        pltpu.sync_copy(row_hbm.at[pl.ds(qbase, QC)], rowbuf)
        pltpu.sync_copy(col_hbm.at[pl.ds(qbase, QC)], colbuf)
        for b in range(NB):
            pltpu.async_copy(hssh.at[rowbuf.at[b]], gbufs[b], semg[b])

        @pl.loop(0, QC, step=NB)
        def ring(j):
            descs = []
            for b in range(NB):
                pltpu.make_async_copy(
                    hssh.at[rowbuf.at[j + b]], gbufs[b], semg[b]).wait()
                descs.append(pltpu.async_copy(
                    gbufs[b], shacc.at[colbuf.at[j + b]], sems[b], add=True))
            for b in range(NB):
                descs[b].wait()

                @pl.when(j + b + NB < QC)
                def _():
                    pltpu.async_copy(
                        hssh.at[rowbuf.at[j + b + NB]], gbufs[b], semg[b])

    plsc.subcore_barrier()
    pltpu.sync_copy(shacc.at[pl.ds(s * ROWS_PT, ROWS_PT)],
                    out_hbm.at[c, pl.ds(s * ROWS_PT, ROWS_PT)])


NH = NP // 2
RPH = NH // 16
EPT = EP // 16
RPT = RPH // 16


@functools.partial(
    pl.kernel,
    out_type=jax.ShapeDtypeStruct((NP // 16, 16), jnp.float32),
    mesh=_mesh,
    compiler_params=pltpu.CompilerParams(needs_layout_passes=False, use_tc_tiling_on_sc=False),
    scratch_types=[
        pltpu.VMEM((NP,), jnp.float32),
        pltpu.VMEM((EPT,), jnp.int32),
        pltpu.VMEM((EPT,), jnp.int32),
        pltpu.VMEM((RPH, 16), jnp.float32),
        pltpu.VMEM((RPT, 16), jnp.float32),
        pltpu.VMEM((RPT, 16), jnp.float32),
        pltpu.VMEM_SHARED((16, RPH, 16), jnp.float32),
    ],
)
def _sc_seg2(ts_hbm, row_hbm, col_hbm, dinv_hbm, x0bc_hbm, out_hbm,
             tsbuf, rowbuf, colbuf, acc2d, fbuf, tbuf, sh16):
    c = lax.axis_index("c")
    s = lax.axis_index("s")
    pltpu.sync_copy(ts_hbm, tsbuf)
    pltpu.sync_copy(row_hbm.at[pl.ds(s * EPT, EPT)], rowbuf)
    pltpu.sync_copy(col_hbm.at[pl.ds(s * EPT, EPT)], colbuf)
    zeros = jnp.zeros((16,), jnp.float32)

    def zbody(i, _):
        acc2d[i, pl.ds(0, 16)] = zeros
        return 0

    lax.fori_loop(0, RPH, zbody, 0)

    half_base = c * NH

    def ebody(i, _):
        sl = pl.ds(i * 16, 16)
        r = rowbuf[sl]
        cc = colbuf[sl]
        v = plsc.load_gather(tsbuf, [r])
        lc = cc - half_base
        valid = (lc >= 0) & (lc < NH)
        slc = jnp.where(valid, lc, 0)
        plsc.addupdate_scatter(
            acc2d,
            [lax.shift_right_logical(slc, 4), lax.bitwise_and(slc, 15)],
            v, mask=valid)
        return 0

    lax.fori_loop(0, EPT // 16, ebody, 0)

    pltpu.sync_copy(acc2d, sh16.at[s])
    plsc.subcore_barrier()
    rbase = s * RPT
    pltpu.sync_copy(sh16.at[0, pl.ds(rbase, RPT)], fbuf)
    for t in range(1, 16):
        pltpu.sync_copy(sh16.at[t, pl.ds(rbase, RPT)], tbuf)
        for g in range(RPT):
            fbuf[g, pl.ds(0, 16)] = (fbuf[g, pl.ds(0, 16)]
                                     + tbuf[g, pl.ds(0, 16)])
    gbase = c * RPH + rbase
    pltpu.sync_copy(dinv_hbm.at[pl.ds(gbase, RPT)], tbuf)
    for g in range(RPT):
        fbuf[g, pl.ds(0, 16)] = fbuf[g, pl.ds(0, 16)] * tbuf[g, pl.ds(0, 16)]
    pltpu.sync_copy(x0bc_hbm.at[pl.ds(gbase, RPT)], tbuf)
    for g in range(RPT):
        fbuf[g, pl.ds(0, 16)] = fbuf[g, pl.ds(0, 16)] + tbuf[g, pl.ds(0, 16)]
    pltpu.sync_copy(fbuf, out_hbm.at[pl.ds(gbase, RPT)])


def _tc_dense1(x_ref, c1w_ref, f1w_ref, f1b_ref, degp_ref,
               hs_ref, x0_ref, dinv_ref):
    deg = degp_ref[...]
    dinv = jnp.where(deg > 0, lax.rsqrt(deg), 0.0)
    x = x_ref[...]
    h = jnp.dot(x, c1w_ref[...], preferred_element_type=jnp.float32)
    hsc = h * dinv
    hs_ref[0] = hsc[:, :DH]
    hs_ref[1] = hsc[:, DH:]
    x0_ref[...] = jnp.maximum(
        jnp.dot(x, f1w_ref[...], preferred_element_type=jnp.float32)
        + f1b_ref[...], 0.0)
    dinv_ref[...] = dinv


def _tc_mid(acc_ref, x0_ref, dinv_ref, c1b_ref, c2w_ref, f2w_ref, f2b_ref,
            c2b_ref, ts_ref, x0b_ref):
    agg = jnp.concatenate([acc_ref[0], acc_ref[1]], axis=-1)
    dinv = dinv_ref[...]
    conv1 = jnp.maximum(agg * dinv + c1b_ref[...], 0.0)
    xsum = x0_ref[...] + conv1
    ts_ref[...] = jnp.dot(xsum, c2w_ref[...],
                          preferred_element_type=jnp.float32) * dinv
    x0b_ref[...] = (jnp.dot(xsum, f2w_ref[...],
                            preferred_element_type=jnp.float32)
                    + f2b_ref[...] + c2b_ref[...])


def kernel(x, edge_index, conv1_W, conv1_b, conv2_W, conv2_b,
           fc1_W, fc1_b, fc2_W, fc2_b):
    f32 = jnp.float32
    xp = jnp.pad(x, ((0, NP - N), (0, 0)))
    fill = jnp.full((EP - E,), NP - 1, dtype=jnp.int32)
    rowp = jnp.concatenate([edge_index[0], fill])
    colp = jnp.concatenate([edge_index[1], fill])
    row2d = rowp.reshape(NCHUNK, CHUNK)
    col2d = colp.reshape(NCHUNK, CHUNK)

    degp = _sc_degree(colp).reshape(NP, 1)

    hs, x0, dinv = pl.pallas_call(
        _tc_dense1,
        grid=(RB,),
        in_specs=[
            pl.BlockSpec((1024, D), lambda i: (i, 0)),
            pl.BlockSpec((D, D), lambda i: (0, 0)),
            pl.BlockSpec((D, D), lambda i: (0, 0)),
            pl.BlockSpec((1, D), lambda i: (0, 0)),
            pl.BlockSpec((1024, 1), lambda i: (i, 0)),
        ],
        out_specs=[
            pl.BlockSpec((2, 1024, DH), lambda i: (0, i, 0)),
            pl.BlockSpec((1024, D), lambda i: (i, 0)),
            pl.BlockSpec((1024, 1), lambda i: (i, 0)),
        ],
        out_shape=[
            jax.ShapeDtypeStruct((2, NP, DH), f32),
            jax.ShapeDtypeStruct((NP, D), f32),
            jax.ShapeDtypeStruct((NP, 1), f32),
        ],
    )(xp, conv1_W, fc1_W, fc1_b.reshape(1, D), degp)

    acc2 = _sc_agg(hs, row2d, col2d)

    ts, x0b = pl.pallas_call(
        _tc_mid,
        grid=(RB,),
        in_specs=[
            pl.BlockSpec((2, 1024, DH), lambda i: (0, i, 0)),
            pl.BlockSpec((1024, D), lambda i: (i, 0)),
            pl.BlockSpec((1024, 1), lambda i: (i, 0)),
            pl.BlockSpec((1, D), lambda i: (0, 0)),
            pl.BlockSpec((D, 1), lambda i: (0, 0)),
            pl.BlockSpec((D, 1), lambda i: (0, 0)),
            pl.BlockSpec((1, 1), lambda i: (0, 0)),
            pl.BlockSpec((1, 1), lambda i: (0, 0)),
        ],
        out_specs=[
            pl.BlockSpec((1024, 1), lambda i: (i, 0)),
            pl.BlockSpec((1024, 1), lambda i: (i, 0)),
        ],
        out_shape=[
            jax.ShapeDtypeStruct((NP, 1), f32),
            jax.ShapeDtypeStruct((NP, 1), f32),
        ],
    )(acc2, x0, dinv, conv1_b.reshape(1, D), conv2_W, fc2_W,
      fc2_b.reshape(1, 1), conv2_b.reshape(1, 1))

    out2d = _sc_seg2(ts.reshape(NP), rowp, colp,
                     dinv.reshape(NP // 16, 16), x0b.reshape(NP // 16, 16))
    return out2d.reshape(NP, 1)[:N]


# speedup vs baseline: 1.0469x; 1.0375x over previous
import functools

import jax
import jax.numpy as jnp
from jax import lax
from jax.experimental import pallas as pl
from jax.experimental.pallas import tpu as pltpu
from jax.experimental.pallas import tpu_sc as plsc

N = 10000
D = 128
E = 320000
NP = 10240
EP = 327680
CHUNK = 128
NCHUNK = EP // CHUNK
NW = 32
CPW = NCHUNK // NW
EPW = EP // NW
ROWS_PT = NP // 16
RB = NP // 1024

_mesh = plsc.VectorSubcoreMesh(core_axis_name="c", subcore_axis_name="s")


@functools.partial(
    pl.kernel,
    out_type=jax.ShapeDtypeStruct((NW, NP), jnp.float32),
    mesh=_mesh,
    compiler_params=pltpu.CompilerParams(needs_layout_passes=False, use_tc_tiling_on_sc=False),
    scratch_types=[
        pltpu.VMEM((EPW,), jnp.int32),
        pltpu.VMEM((NP,), jnp.float32),
    ],
)
def _sc_degree(col_hbm, out_hbm, colbuf, acc):
    w = lax.axis_index("s") * 2 + lax.axis_index("c")
    pltpu.sync_copy(col_hbm.at[pl.ds(w * EPW, EPW)], colbuf)
    zeros = jnp.zeros((16,), jnp.float32)

    def zbody(i, _):
        acc[pl.ds(i * 16, 16)] = zeros
        return 0

    lax.fori_loop(0, NP // 16, zbody, 0)
    ones = jnp.ones((16,), jnp.float32)

    def ebody(i, _):
        idx = colbuf[pl.ds(i * 16, 16)]
        plsc.addupdate_scatter(acc, [idx], ones)
        return 0

    lax.fori_loop(0, EPW // 16, ebody, 0)
    pltpu.sync_copy(acc, out_hbm.at[w])


DH = D // 2
CPT = NCHUNK // 16
CS = 1


NB = 4
QC = 40


@functools.partial(
    pl.kernel,
    out_type=jax.ShapeDtypeStruct((2, NP, DH), jnp.float32),
    mesh=_mesh,
    compiler_params=pltpu.CompilerParams(needs_layout_passes=False, use_tc_tiling_on_sc=False),
    scratch_types=[
        pltpu.VMEM((QC, CHUNK), jnp.int32),
        pltpu.VMEM((QC, CHUNK), jnp.int32),
        [pltpu.VMEM((CHUNK, DH), jnp.float32)] * NB,
        pltpu.VMEM_SHARED((NP, DH), jnp.float32),
        pltpu.VMEM_SHARED((NP, DH), jnp.float32),
        [pltpu.SemaphoreType.DMA] * NB,
        [pltpu.SemaphoreType.DMA] * NB,
    ],
)
def _sc_agg(hs_hbm, row_hbm, col_hbm, out_hbm, rowbuf, colbuf, gbufs,
            hssh, shacc, semg, sems):
    gb0 = gbufs[0]
    gb1 = gbufs[1]
    c = lax.axis_index("c")
    s = lax.axis_index("s")

    zeros = jnp.zeros((16,), jnp.float32)

    def zb(i, _):
        r = i // 4
        k = i - r * 4
        gb0[r, pl.ds(k * 16, 16)] = zeros
        return 0

    lax.fori_loop(0, CHUNK * (DH // 16), zb, 0)
    pltpu.sync_copy(hs_hbm.at[c].at[pl.ds(s * ROWS_PT, ROWS_PT)],
                    hssh.at[pl.ds(s * ROWS_PT, ROWS_PT)])
    for t in range(ROWS_PT // CHUNK):
        base = s * ROWS_PT + t * CHUNK
        pltpu.sync_copy(gb0, shacc.at[pl.ds(base, CHUNK)])
    plsc.subcore_barrier()

    for q in range(CPT // QC):
        qbase = s * CPT + q * QC
        pltpu.sync_copy(row_hbm.at[pl.ds(qbase, QC)], rowbuf)
        pltpu.sync_copy(col_hbm.at[pl.ds(qbase, QC)], colbuf)
        for b in range(NB):
            pltpu.async_copy(hssh.at[rowbuf.at[b]], gbufs[b], semg[b])

        @pl.loop(0, QC, step=NB)
        def ring(j):
            descs = []
            for b in range(NB):
                pltpu.make_async_copy(
                    hssh.at[rowbuf.at[j + b]], gbufs[b], semg[b]).wait()
                descs.append(pltpu.async_copy(
                    gbufs[b], shacc.at[colbuf.at[j + b]], sems[b], add=True))
            for b in range(NB):
                descs[b].wait()

                @pl.when(j + b + NB < QC)
                def _():
                    pltpu.async_copy(
                        hssh.at[rowbuf.at[j + b + NB]], gbufs[b], semg[b])

    plsc.subcore_barrier()
    pltpu.sync_copy(shacc.at[pl.ds(s * ROWS_PT, ROWS_PT)],
                    out_hbm.at[c, pl.ds(s * ROWS_PT, ROWS_PT)])


NH = NP // 2
RPH = NH // 16
EPT = EP // 16
RPT = RPH // 16


@functools.partial(
    pl.kernel,
    out_type=jax.ShapeDtypeStruct((NP // 16, 16), jnp.float32),
    mesh=_mesh,
    compiler_params=pltpu.CompilerParams(needs_layout_passes=False, use_tc_tiling_on_sc=False),
    scratch_types=[
        pltpu.VMEM((NP,), jnp.float32),
        pltpu.VMEM((EPT,), jnp.int32),
        pltpu.VMEM((EPT,), jnp.int32),
        pltpu.VMEM((RPH, 16), jnp.float32),
        pltpu.VMEM((RPT, 16), jnp.float32),
        pltpu.VMEM((RPT, 16), jnp.float32),
        pltpu.VMEM_SHARED((16, RPH, 16), jnp.float32),
    ],
)
def _sc_seg2(ts_hbm, row_hbm, col_hbm, dinv_hbm, x0bc_hbm, out_hbm,
             tsbuf, rowbuf, colbuf, acc2d, fbuf, tbuf, sh16):
    c = lax.axis_index("c")
    s = lax.axis_index("s")
    pltpu.sync_copy(ts_hbm, tsbuf)
    pltpu.sync_copy(row_hbm.at[pl.ds(s * EPT, EPT)], rowbuf)
    pltpu.sync_copy(col_hbm.at[pl.ds(s * EPT, EPT)], colbuf)
    zeros = jnp.zeros((16,), jnp.float32)

    def zbody(i, _):
        acc2d[i, pl.ds(0, 16)] = zeros
        return 0

    lax.fori_loop(0, RPH, zbody, 0)

    half_base = c * NH

    def ebody(i, _):
        sl = pl.ds(i * 16, 16)
        r = rowbuf[sl]
        cc = colbuf[sl]
        v = plsc.load_gather(tsbuf, [r])
        lc = cc - half_base
        valid = (lc >= 0) & (lc < NH)
        slc = jnp.where(valid, lc, 0)
        plsc.addupdate_scatter(
            acc2d,
            [lax.shift_right_logical(slc, 4), lax.bitwise_and(slc, 15)],
            v, mask=valid)
        return 0

    lax.fori_loop(0, EPT // 16, ebody, 0)

    pltpu.sync_copy(acc2d, sh16.at[s])
    plsc.subcore_barrier()
    rbase = s * RPT
    pltpu.sync_copy(sh16.at[0, pl.ds(rbase, RPT)], fbuf)
    for t in range(1, 16):
        pltpu.sync_copy(sh16.at[t, pl.ds(rbase, RPT)], tbuf)
        for g in range(RPT):
            fbuf[g, pl.ds(0, 16)] = (fbuf[g, pl.ds(0, 16)]
                                     + tbuf[g, pl.ds(0, 16)])
    gbase = c * RPH + rbase
    pltpu.sync_copy(dinv_hbm.at[pl.ds(gbase, RPT)], tbuf)
    for g in range(RPT):
        fbuf[g, pl.ds(0, 16)] = fbuf[g, pl.ds(0, 16)] * tbuf[g, pl.ds(0, 16)]
    pltpu.sync_copy(x0bc_hbm.at[pl.ds(gbase, RPT)], tbuf)
    for g in range(RPT):
        fbuf[g, pl.ds(0, 16)] = fbuf[g, pl.ds(0, 16)] + tbuf[g, pl.ds(0, 16)]
    pltpu.sync_copy(fbuf, out_hbm.at[pl.ds(gbase, RPT)])


def _tc_dense1(x_ref, c1w_ref, f1w_ref, f1b_ref, degp_ref,
               hs_ref, x0_ref, dinv_ref):
    deg = jnp.sum(degp_ref[...], axis=0)
    dinv = jnp.where(deg > 0, lax.rsqrt(deg), 0.0)
    x = x_ref[...]
    h = jnp.dot(x, c1w_ref[...], preferred_element_type=jnp.float32)
    hsc = h * dinv[:, None]
    hs_ref[0] = hsc[:, :DH]
    hs_ref[1] = hsc[:, DH:]
    x0_ref[...] = jnp.maximum(
        jnp.dot(x, f1w_ref[...], preferred_element_type=jnp.float32)
        + f1b_ref[...], 0.0)
    dinv_ref[...] = dinv[:, None]


def _tc_mid(acc_ref, x0_ref, dinv_ref, c1b_ref, c2w_ref, f2w_ref, f2b_ref,
            c2b_ref, ts_ref, x0b_ref):
    agg = jnp.concatenate([acc_ref[0], acc_ref[1]], axis=-1)
    dinv = dinv_ref[...]
    conv1 = jnp.maximum(agg * dinv + c1b_ref[...], 0.0)
    xsum = x0_ref[...] + conv1
    ts_ref[...] = jnp.dot(xsum, c2w_ref[...],
                          preferred_element_type=jnp.float32) * dinv
    x0b_ref[...] = (jnp.dot(xsum, f2w_ref[...],
                            preferred_element_type=jnp.float32)
                    + f2b_ref[...] + c2b_ref[...])


def kernel(x, edge_index, conv1_W, conv1_b, conv2_W, conv2_b,
           fc1_W, fc1_b, fc2_W, fc2_b):
    f32 = jnp.float32
    xp = jnp.pad(x, ((0, NP - N), (0, 0)))
    fill = jnp.full((EP - E,), NP - 1, dtype=jnp.int32)
    rowp = jnp.concatenate([edge_index[0], fill])
    colp = jnp.concatenate([edge_index[1], fill])
    row2d = rowp.reshape(NCHUNK, CHUNK)
    col2d = colp.reshape(NCHUNK, CHUNK)

    degp = _sc_degree(colp)

    hs, x0, dinv = pl.pallas_call(
        _tc_dense1,
        grid=(RB,),
        in_specs=[
            pl.BlockSpec((1024, D), lambda i: (i, 0)),
            pl.BlockSpec((D, D), lambda i: (0, 0)),
            pl.BlockSpec((D, D), lambda i: (0, 0)),
            pl.BlockSpec((1, D), lambda i: (0, 0)),
            pl.BlockSpec((NW, 1024), lambda i: (0, i)),
        ],
        out_specs=[
            pl.BlockSpec((2, 1024, DH), lambda i: (0, i, 0)),
            pl.BlockSpec((1024, D), lambda i: (i, 0)),
            pl.BlockSpec((1024, 1), lambda i: (i, 0)),
        ],
        out_shape=[
            jax.ShapeDtypeStruct((2, NP, DH), f32),
            jax.ShapeDtypeStruct((NP, D), f32),
            jax.ShapeDtypeStruct((NP, 1), f32),
        ],
    )(xp, conv1_W, fc1_W, fc1_b.reshape(1, D), degp)

    acc2 = _sc_agg(hs, row2d, col2d)

    ts, x0b = pl.pallas_call(
        _tc_mid,
        grid=(RB,),
        in_specs=[
            pl.BlockSpec((2, 1024, DH), lambda i: (0, i, 0)),
            pl.BlockSpec((1024, D), lambda i: (i, 0)),
            pl.BlockSpec((1024, 1), lambda i: (i, 0)),
            pl.BlockSpec((1, D), lambda i: (0, 0)),
            pl.BlockSpec((D, 1), lambda i: (0, 0)),
            pl.BlockSpec((D, 1), lambda i: (0, 0)),
            pl.BlockSpec((1, 1), lambda i: (0, 0)),
            pl.BlockSpec((1, 1), lambda i: (0, 0)),
        ],
        out_specs=[
            pl.BlockSpec((1024, 1), lambda i: (i, 0)),
            pl.BlockSpec((1024, 1), lambda i: (i, 0)),
        ],
        out_shape=[
            jax.ShapeDtypeStruct((NP, 1), f32),
            jax.ShapeDtypeStruct((NP, 1), f32),
        ],
    )(acc2, x0, dinv, conv1_b.reshape(1, D), conv2_W, fc2_W,
      fc2_b.reshape(1, 1), conv2_b.reshape(1, 1))

    out2d = _sc_seg2(ts.reshape(NP), rowp, colp,
                     dinv.reshape(NP // 16, 16), x0b.reshape(NP // 16, 16))
    return out2d.reshape(NP, 1)[:N]
